# baseline (speedup 1.0000x reference)
import functools

import jax
import jax.numpy as jnp
from jax import lax
from jax.experimental import pallas as pl
from jax.experimental.pallas import tpu as pltpu
from jax.experimental.pallas import tpu_sc as plsc

T, D, F, E = 2048, 2048, 8192, 8
M = 512
FB = 512
NFB = F // FB
VISITS = T // M + E
NSTEPS = VISITS * NFB
R_MAX = VISITS * M

NW = 32
TPW = T // NW
CH = 32
NCH = TPW // CH


def _route_body(x_ref, wr_ref, br_ref, pos_ref, ntiles_ref):
    x = x_ref[...]
    keys = jnp.dot(x, wr_ref[...], preferred_element_type=jnp.float32)
    keys = keys + br_ref[0, 0]
    e = jnp.remainder(jnp.floor(keys).astype(jnp.int32), E)
    lane = lax.broadcasted_iota(jnp.int32, (T, E), 1)
    onehot = (e == lane).astype(jnp.float32)

    r_i = lax.broadcasted_iota(jnp.int32, (T, T), 0)
    c_i = lax.broadcasted_iota(jnp.int32, (T, T), 1)
    tril = (r_i > c_i).astype(jnp.float32)
    prior = jnp.dot(tril, onehot, preferred_element_type=jnp.float32)
    rank = jnp.sum(prior * onehot, axis=1)

    counts = jnp.sum(onehot, axis=0, keepdims=True)
    ptiles = jnp.floor((counts + (M - 1)) * (1.0 / M))
    a_i = lax.broadcasted_iota(jnp.int32, (E, E), 0)
    b_i = lax.broadcasted_iota(jnp.int32, (E, E), 1)
    triu = (a_i < b_i).astype(jnp.float32)
    pad_off = jnp.dot(ptiles, triu, preferred_element_type=jnp.float32) * M

    pos = rank + jnp.sum(onehot * pad_off, axis=1)
    pos_ref[...] = pos.astype(jnp.int32).reshape(1, T)
    ntiles_ref[...] = ptiles.astype(jnp.int32)


def _route(x2d, wr, br2d):
    return pl.pallas_call(
        _route_body,
        out_shape=(
            jax.ShapeDtypeStruct((1, T), jnp.int32),
            jax.ShapeDtypeStruct((1, E), jnp.int32),
        ),
    )(x2d, wr, br2d)


@functools.lru_cache(maxsize=1)
def _sc_kernels():
    mesh = plsc.VectorSubcoreMesh(core_axis_name="c", subcore_axis_name="s")
    scratch = [
        pltpu.VMEM((NCH, CH), jnp.int32),
        pltpu.VMEM((CH, D), jnp.float32),
        pltpu.SemaphoreType.DMA,
    ]

    @functools.partial(
        pl.kernel,
        out_type=jax.ShapeDtypeStruct((R_MAX, D), jnp.float32),
        mesh=mesh,
        scratch_types=scratch,
    )
    def sc_scatter(x_hbm, pos_hbm, xs_hbm, idx_v, rows_v, sem):
        wid = lax.axis_index("s") * 2 + lax.axis_index("c")
        pltpu.sync_copy(pos_hbm.at[wid], idx_v)
        for j in range(NCH):
            pltpu.sync_copy(x_hbm.at[pl.ds(wid * TPW + j * CH, CH)], rows_v)
            pltpu.async_copy(rows_v, xs_hbm.at[idx_v.at[j]], sem).wait()

    @functools.partial(
        pl.kernel,
        out_type=jax.ShapeDtypeStruct((T, D), jnp.float32),
        mesh=mesh,
        scratch_types=scratch,
    )
    def sc_gather(ys_hbm, pos_hbm, out_hbm, idx_v, rows_v, sem):
        wid = lax.axis_index("s") * 2 + lax.axis_index("c")
        pltpu.sync_copy(pos_hbm.at[wid], idx_v)
        for j in range(NCH):
            pltpu.async_copy(ys_hbm.at[idx_v.at[j]], rows_v, sem).wait()
            pltpu.sync_copy(rows_v, out_hbm.at[pl.ds(wid * TPW + j * CH, CH)])

    return sc_scatter, sc_gather


def _ffn_body(sched_ref, xs_ref, w1_ref, b1_ref, w2_ref, b2_ref, out_ref):
    s = pl.program_id(0)
    act = sched_ref[4, s] == 1
    init = sched_ref[3, s] == 1

    @pl.when(act)
    def _():
        xb = xs_ref[...].astype(jnp.bfloat16)
        w1b = w1_ref[0].astype(jnp.bfloat16)
        h = jnp.dot(xb, w1b,
                    preferred_element_type=jnp.float32) + b1_ref[0, 0][None, :]
        h = jnp.maximum(h, 0.0).astype(jnp.bfloat16)
        w2b = w2_ref[0].astype(jnp.bfloat16)
        contrib = jnp.dot(h, w2b, preferred_element_type=jnp.float32)

        @pl.when(init)
        def _():
            out_ref[...] = contrib + b2_ref[0, 0][None, :]

        @pl.when(jnp.logical_not(init))
        def _():
            out_ref[...] += contrib


def _ffn(sched, xs, w1, b1_3d, w2, b2_3d):
    grid_spec = pltpu.PrefetchScalarGridSpec(
        num_scalar_prefetch=1,
        grid=(NSTEPS,),
        in_specs=[
            pl.BlockSpec((M, D), lambda s, sr: (sr[1, s], 0)),
            pl.BlockSpec((1, D, FB), lambda s, sr: (sr[0, s], 0, sr[2, s])),
            pl.BlockSpec((1, 1, FB), lambda s, sr: (sr[0, s], 0, sr[2, s])),
            pl.BlockSpec((1, FB, D), lambda s, sr: (sr[0, s], sr[2, s], 0)),
            pl.BlockSpec((1, 1, D), lambda s, sr: (sr[0, s], 0, 0)),
        ],
        out_specs=pl.BlockSpec((M, D), lambda s, sr: (sr[1, s], 0)),
    )
    return pl.pallas_call(
        _ffn_body,
        grid_spec=grid_spec,
        out_shape=jax.ShapeDtypeStruct((R_MAX, D), jnp.float32),
        compiler_params=pltpu.CompilerParams(
            dimension_semantics=("arbitrary",)),
    )(sched, xs, w1, b1_3d, w2, b2_3d)


def _schedule(ntiles):
    tiles_cum = jnp.cumsum(ntiles[0])
    total = tiles_cum[E - 1]
    v_ids = jnp.arange(VISITS, dtype=jnp.int32)
    vmap = jnp.minimum(v_ids, total - 1)
    expert_v = jnp.searchsorted(tiles_cum, vmap, side="right").astype(jnp.int32)
    active_v = (v_ids < total).astype(jnp.int32)

    s_ids = jnp.arange(NSTEPS, dtype=jnp.int32)
    v = s_ids // NFB
    f_raw = s_ids % NFB
    act = active_v[v]
    estep = expert_v[v]
    vstep = vmap[v]
    fstep = jnp.where(act == 1, f_raw, NFB - 1)
    init = ((act == 1) & (f_raw == 0)).astype(jnp.int32)
    return jnp.stack([estep, vstep, fstep, init, act]).astype(jnp.int32)


def kernel(x, Wr, br, W1, b1, W2, b2):
    x2d = x.reshape(T, D)
    br2d = br.reshape(1, 1)
    pos, ntiles = _route(x2d, Wr, br2d)
    sched = _schedule(ntiles)
    pos3 = pos.reshape(NW, NCH, CH)
    sc_scatter, sc_gather = _sc_kernels()
    xs = sc_scatter(x2d, pos3)
    ys = _ffn(sched, xs, W1, b1.reshape(E, 1, F), W2, b2.reshape(E, 1, D))
    out = sc_gather(ys, pos3)
    return out.reshape(x.shape)


# speedup vs baseline: 1.1179x; 1.1179x over previous
import functools

import jax
import jax.numpy as jnp
from jax import lax
from jax.experimental import pallas as pl
from jax.experimental.pallas import tpu as pltpu
from jax.experimental.pallas import tpu_sc as plsc

T, D, F, E = 2048, 2048, 8192, 8
M = 512
FB = 1024
NFB = F // FB
VISITS = T // M + E
NSTEPS = VISITS * NFB
R_MAX = VISITS * M

NW = 32
TPW = T // NW
CH = 32
NCH = TPW // CH


def _route_body(x_ref, wr_ref, br_ref, pos_ref, ntiles_ref):
    x = x_ref[...]
    keys = jnp.dot(x, wr_ref[...], preferred_element_type=jnp.float32)
    keys = keys + br_ref[0, 0]
    e = jnp.remainder(jnp.floor(keys).astype(jnp.int32), E)
    lane = lax.broadcasted_iota(jnp.int32, (T, E), 1)
    onehot = (e == lane).astype(jnp.float32)

    r_i = lax.broadcasted_iota(jnp.int32, (T, T), 0)
    c_i = lax.broadcasted_iota(jnp.int32, (T, T), 1)
    tril = (r_i > c_i).astype(jnp.float32)
    prior = jnp.dot(tril, onehot, preferred_element_type=jnp.float32)
    rank = jnp.sum(prior * onehot, axis=1)

    counts = jnp.sum(onehot, axis=0, keepdims=True)
    ptiles = jnp.floor((counts + (M - 1)) * (1.0 / M))
    a_i = lax.broadcasted_iota(jnp.int32, (E, E), 0)
    b_i = lax.broadcasted_iota(jnp.int32, (E, E), 1)
    triu = (a_i < b_i).astype(jnp.float32)
    pad_off = jnp.dot(ptiles, triu, preferred_element_type=jnp.float32) * M

    pos = rank + jnp.sum(onehot * pad_off, axis=1)
    pos_ref[...] = pos.astype(jnp.int32).reshape(1, T)
    ntiles_ref[...] = ptiles.astype(jnp.int32)


def _route(x2d, wr, br2d):
    return pl.pallas_call(
        _route_body,
        out_shape=(
            jax.ShapeDtypeStruct((1, T), jnp.int32),
            jax.ShapeDtypeStruct((1, E), jnp.int32),
        ),
    )(x2d, wr, br2d)


@functools.lru_cache(maxsize=1)
def _sc_kernels():
    mesh = plsc.VectorSubcoreMesh(core_axis_name="c", subcore_axis_name="s")
    scratch = [
        pltpu.VMEM((NCH, CH), jnp.int32),
        pltpu.VMEM((CH, D), jnp.float32),
        pltpu.SemaphoreType.DMA,
    ]

    @functools.partial(
        pl.kernel,
        out_type=jax.ShapeDtypeStruct((R_MAX, D), jnp.float32),
        mesh=mesh,
        scratch_types=scratch,
    )
    def sc_scatter(x_hbm, pos_hbm, xs_hbm, idx_v, rows_v, sem):
        wid = lax.axis_index("s") * 2 + lax.axis_index("c")
        pltpu.sync_copy(pos_hbm.at[wid], idx_v)
        for j in range(NCH):
            pltpu.sync_copy(x_hbm.at[pl.ds(wid * TPW + j * CH, CH)], rows_v)
            pltpu.async_copy(rows_v, xs_hbm.at[idx_v.at[j]], sem).wait()

    @functools.partial(
        pl.kernel,
        out_type=jax.ShapeDtypeStruct((T, D), jnp.float32),
        mesh=mesh,
        scratch_types=scratch,
    )
    def sc_gather(ys_hbm, pos_hbm, out_hbm, idx_v, rows_v, sem):
        wid = lax.axis_index("s") * 2 + lax.axis_index("c")
        pltpu.sync_copy(pos_hbm.at[wid], idx_v)
        for j in range(NCH):
            pltpu.async_copy(ys_hbm.at[idx_v.at[j]], rows_v, sem).wait()
            pltpu.sync_copy(rows_v, out_hbm.at[pl.ds(wid * TPW + j * CH, CH)])

    return sc_scatter, sc_gather


def _ffn_body(sched_ref, xs_ref, w1_ref, b1_ref, w2_ref, b2_ref, out_ref):
    s = pl.program_id(0)
    act = sched_ref[4, s] == 1
    init = sched_ref[3, s] == 1

    @pl.when(act)
    def _():
        h = jnp.dot(xs_ref[...], w1_ref[0],
                    preferred_element_type=jnp.float32) + b1_ref[0, 0][None, :]
        h = jnp.maximum(h, 0.0)
        contrib = jnp.dot(h, w2_ref[0], preferred_element_type=jnp.float32)

        @pl.when(init)
        def _():
            out_ref[...] = contrib + b2_ref[0, 0][None, :]

        @pl.when(jnp.logical_not(init))
        def _():
            out_ref[...] += contrib


def _ffn(sched, xs, w1, b1_3d, w2, b2_3d):
    grid_spec = pltpu.PrefetchScalarGridSpec(
        num_scalar_prefetch=1,
        grid=(NSTEPS,),
        in_specs=[
            pl.BlockSpec((M, D), lambda s, sr: (sr[1, s], 0)),
            pl.BlockSpec((1, D, FB), lambda s, sr: (sr[0, s], 0, sr[2, s])),
            pl.BlockSpec((1, 1, FB), lambda s, sr: (sr[0, s], 0, sr[2, s])),
            pl.BlockSpec((1, FB, D), lambda s, sr: (sr[0, s], sr[2, s], 0)),
            pl.BlockSpec((1, 1, D), lambda s, sr: (sr[0, s], 0, 0)),
        ],
        out_specs=pl.BlockSpec((M, D), lambda s, sr: (sr[1, s], 0)),
    )
    return pl.pallas_call(
        _ffn_body,
        grid_spec=grid_spec,
        out_shape=jax.ShapeDtypeStruct((R_MAX, D), jnp.float32),
        compiler_params=pltpu.CompilerParams(
            dimension_semantics=("arbitrary",)),
    )(sched, xs, w1, b1_3d, w2, b2_3d)


def _schedule(ntiles):
    tiles_cum = jnp.cumsum(ntiles[0])
    total = tiles_cum[E - 1]
    v_ids = jnp.arange(VISITS, dtype=jnp.int32)
    vmap = jnp.minimum(v_ids, total - 1)
    expert_v = jnp.searchsorted(tiles_cum, vmap, side="right").astype(jnp.int32)
    active_v = (v_ids < total).astype(jnp.int32)

    s_ids = jnp.arange(NSTEPS, dtype=jnp.int32)
    v = s_ids // NFB
    f_raw = s_ids % NFB
    act = active_v[v]
    estep = expert_v[v]
    vstep = vmap[v]
    fstep = jnp.where(act == 1, f_raw, NFB - 1)
    init = ((act == 1) & (f_raw == 0)).astype(jnp.int32)
    return jnp.stack([estep, vstep, fstep, init, act]).astype(jnp.int32)


def kernel(x, Wr, br, W1, b1, W2, b2):
    x2d = x.reshape(T, D)
    br2d = br.reshape(1, 1)
    pos, ntiles = _route(x2d, Wr, br2d)
    sched = _schedule(ntiles)
    pos3 = pos.reshape(NW, NCH, CH)
    sc_scatter, sc_gather = _sc_kernels()
    xs = sc_scatter(x2d, pos3)
    ys = _ffn(sched, xs, W1, b1.reshape(E, 1, F), W2, b2.reshape(E, 1, D))
    out = sc_gather(ys, pos3)
    return out.reshape(x.shape)


# speedup vs baseline: 1.1629x; 1.0403x over previous
import functools

import jax
import jax.numpy as jnp
from jax import lax
from jax.experimental import pallas as pl
from jax.experimental.pallas import tpu as pltpu
from jax.experimental.pallas import tpu_sc as plsc

T, D, F, E = 2048, 2048, 8192, 8
M = 512
FB = 1024
NFB = F // FB
VISITS = T // M + E
NSTEPS = VISITS * NFB
R_MAX = VISITS * M

NW = 32
TPW = T // NW
CH = 32
NCH = TPW // CH


def _route_body(x_ref, wr_ref, br_ref, pos_ref, sched_ref):
    x = x_ref[...]
    keys = jnp.dot(x, wr_ref[...], preferred_element_type=jnp.float32)
    keys = keys + br_ref[0, 0]
    e = jnp.remainder(jnp.floor(keys).astype(jnp.int32), E)
    lane = lax.broadcasted_iota(jnp.int32, (T, E), 1)
    onehot = (e == lane).astype(jnp.float32)

    r_i = lax.broadcasted_iota(jnp.int32, (T, T), 0)
    c_i = lax.broadcasted_iota(jnp.int32, (T, T), 1)
    tril = (r_i > c_i).astype(jnp.float32)
    prior = jnp.dot(tril, onehot, preferred_element_type=jnp.float32)
    rank = jnp.sum(prior * onehot, axis=1)

    counts = jnp.sum(onehot, axis=0, keepdims=True)
    ptiles = jnp.floor((counts + (M - 1)) * (1.0 / M))
    a_i = lax.broadcasted_iota(jnp.int32, (E, E), 0)
    b_i = lax.broadcasted_iota(jnp.int32, (E, E), 1)
    triu = (a_i < b_i).astype(jnp.float32)
    pad_off = jnp.dot(ptiles, triu, preferred_element_type=jnp.float32) * M

    pos = rank + jnp.sum(onehot * pad_off, axis=1)
    pos_ref[...] = pos.astype(jnp.int32).reshape(1, T)

    triu_i = (a_i <= b_i).astype(jnp.float32)
    tiles_cum = jnp.dot(ptiles, triu_i, preferred_element_type=jnp.float32)
    total = tiles_cum[0, E - 1].astype(jnp.int32)
    cum_b = jnp.broadcast_to(tiles_cum.astype(jnp.int32), (NSTEPS, E))
    s2 = lax.broadcasted_iota(jnp.int32, (NSTEPS, E), 0)
    v2 = s2 // NFB
    vm2 = jnp.minimum(v2, total - 1)
    estep = jnp.sum((cum_b <= vm2).astype(jnp.int32), axis=1)
    s1 = lax.broadcasted_iota(jnp.int32, (1, NSTEPS), 1)
    v1 = s1 // NFB
    f_raw = s1 - v1 * NFB
    act = (v1 < total).astype(jnp.int32)
    vstep = jnp.minimum(v1, total - 1)
    fstep = jnp.where(act == 1, f_raw, NFB - 1)
    init = jnp.where((act == 1) & (f_raw == 0), 1, 0)
    sched_ref[0:1, :] = estep.reshape(1, NSTEPS)
    sched_ref[1:2, :] = vstep
    sched_ref[2:3, :] = fstep
    sched_ref[3:4, :] = init
    sched_ref[4:5, :] = act
    sched_ref[5:6, :] = jnp.zeros((1, NSTEPS), jnp.int32)
    sched_ref[6:7, :] = jnp.zeros((1, NSTEPS), jnp.int32)
    sched_ref[7:8, :] = jnp.zeros((1, NSTEPS), jnp.int32)


def _route(x2d, wr, br2d):
    return pl.pallas_call(
        _route_body,
        out_shape=(
            jax.ShapeDtypeStruct((1, T), jnp.int32),
            jax.ShapeDtypeStruct((8, NSTEPS), jnp.int32),
        ),
    )(x2d, wr, br2d)


@functools.lru_cache(maxsize=1)
def _sc_kernels():
    mesh = plsc.VectorSubcoreMesh(core_axis_name="c", subcore_axis_name="s")
    scratch = [
        pltpu.VMEM((NCH, CH), jnp.int32),
        pltpu.VMEM((CH, D), jnp.float32),
        pltpu.SemaphoreType.DMA,
    ]

    @functools.partial(
        pl.kernel,
        out_type=jax.ShapeDtypeStruct((R_MAX, D), jnp.float32),
        mesh=mesh,
        scratch_types=scratch,
    )
    def sc_scatter(x_hbm, pos_hbm, xs_hbm, idx_v, rows_v, sem):
        wid = lax.axis_index("s") * 2 + lax.axis_index("c")
        pltpu.sync_copy(pos_hbm.at[wid], idx_v)
        for j in range(NCH):
            pltpu.sync_copy(x_hbm.at[pl.ds(wid * TPW + j * CH, CH)], rows_v)
            pltpu.async_copy(rows_v, xs_hbm.at[idx_v.at[j]], sem).wait()

    @functools.partial(
        pl.kernel,
        out_type=jax.ShapeDtypeStruct((T, D), jnp.float32),
        mesh=mesh,
        scratch_types=scratch,
    )
    def sc_gather(ys_hbm, pos_hbm, out_hbm, idx_v, rows_v, sem):
        wid = lax.axis_index("s") * 2 + lax.axis_index("c")
        pltpu.sync_copy(pos_hbm.at[wid], idx_v)
        for j in range(NCH):
            pltpu.async_copy(ys_hbm.at[idx_v.at[j]], rows_v, sem).wait()
            pltpu.sync_copy(rows_v, out_hbm.at[pl.ds(wid * TPW + j * CH, CH)])

    return sc_scatter, sc_gather


def _ffn_body(sched_ref, xs_ref, w1_ref, b1_ref, w2_ref, b2_ref, out_ref):
    s = pl.program_id(0)
    act = sched_ref[4, s] == 1
    init = sched_ref[3, s] == 1

    @pl.when(act)
    def _():
        h = jnp.dot(xs_ref[...], w1_ref[0],
                    preferred_element_type=jnp.float32) + b1_ref[0, 0][None, :]
        h = jnp.maximum(h, 0.0)
        contrib = jnp.dot(h, w2_ref[0], preferred_element_type=jnp.float32)

        @pl.when(init)
        def _():
            out_ref[...] = contrib + b2_ref[0, 0][None, :]

        @pl.when(jnp.logical_not(init))
        def _():
            out_ref[...] += contrib


def _ffn(sched, xs, w1, b1_3d, w2, b2_3d):
    grid_spec = pltpu.PrefetchScalarGridSpec(
        num_scalar_prefetch=1,
        grid=(NSTEPS,),
        in_specs=[
            pl.BlockSpec((M, D), lambda s, sr: (sr[1, s], 0)),
            pl.BlockSpec((1, D, FB), lambda s, sr: (sr[0, s], 0, sr[2, s])),
            pl.BlockSpec((1, 1, FB), lambda s, sr: (sr[0, s], 0, sr[2, s])),
            pl.BlockSpec((1, FB, D), lambda s, sr: (sr[0, s], sr[2, s], 0)),
            pl.BlockSpec((1, 1, D), lambda s, sr: (sr[0, s], 0, 0)),
        ],
        out_specs=pl.BlockSpec((M, D), lambda s, sr: (sr[1, s], 0)),
    )
    return pl.pallas_call(
        _ffn_body,
        grid_spec=grid_spec,
        out_shape=jax.ShapeDtypeStruct((R_MAX, D), jnp.float32),
        compiler_params=pltpu.CompilerParams(
            dimension_semantics=("arbitrary",)),
    )(sched, xs, w1, b1_3d, w2, b2_3d)


def kernel(x, Wr, br, W1, b1, W2, b2):
    x2d = x.reshape(T, D)
    br2d = br.reshape(1, 1)
    pos, sched = _route(x2d, Wr, br2d)
    pos3 = pos.reshape(NW, NCH, CH)
    sc_scatter, sc_gather = _sc_kernels()
    xs = sc_scatter(x2d, pos3)
    ys = _ffn(sched, xs, W1, b1.reshape(E, 1, F), W2, b2.reshape(E, 1, D))
    out = sc_gather(ys, pos3)
    return out.reshape(x.shape)
